# Initial kernel scaffold; baseline (speedup 1.0000x reference)
import jax
import jax.numpy as jnp
from jax.experimental import pallas as pl


def kernel(X, edge_index, edge_vals, W_w, W_b, prelu_a):
    raise NotImplementedError("write your pallas kernel here")


# speedup vs baseline: 1.9958x; 1.9958x over previous
import functools

import jax
import jax.numpy as jnp
from jax import lax
from jax.experimental import pallas as pl
from jax.experimental.pallas import tpu as pltpu
from jax.experimental.pallas import tpu_sc as plsc

_N = 10000
_E = 320000
_IN = 128
_HID = 128
_K1 = 4
_YW = _K1 * _HID
_NP = 2
_NSPLIT = 5120
_TRASH = 64
_ACC_R = 5248

_NC = 2
_NS = 16
_NW = _NC * _NS
_CHUNK = 128
_EPW = 10240
_E_PAD = _NW * _EPW
_NCHUNKS = _EPW // _CHUNK
_N_PAD = 10240
_RPT = _N_PAD // _NS
_ZR = _ACC_R // _NS


def _matmul_body(x_ref, w_ref, y_ref):
    y_ref[...] = jnp.dot(x_ref[...], w_ref[...],
                         preferred_element_type=jnp.float32)


def _compute_y(x, wcat):
    blk = 1000
    return pl.pallas_call(
        _matmul_body,
        grid=(_N // blk,),
        in_specs=[
            pl.BlockSpec((blk, _IN), lambda i: (i, 0)),
            pl.BlockSpec((_IN, _YW), lambda i: (0, 0)),
        ],
        out_specs=pl.BlockSpec((blk, _YW), lambda i: (i, 0)),
        out_shape=jax.ShapeDtypeStruct((_N, _YW), jnp.float32),
    )(x, wcat)


def _sc_msg_kernel(y_hbm, col_hbm, vals_hbm, msgs_hbm,
                   colv, valsv, rows_v, msg_v, sem):
    c = lax.axis_index("c")
    s = lax.axis_index("s")
    wid = s * _NC + c
    base0 = wid * _EPW

    def chunk_body(ci, carry):
        base = base0 + ci * _CHUNK
        pltpu.sync_copy(col_hbm.at[pl.ds(base, _CHUNK)], colv)
        gather = pltpu.async_copy(y_hbm.at[colv], rows_v, sem)
        for k in range(_K1):
            pltpu.sync_copy(vals_hbm.at[pl.ds(k * _E_PAD + base, _CHUNK)],
                            valsv.at[k])
        gather.wait()

        for gi in range(_CHUNK // 16):
            vv = [valsv[k, pl.ds(gi * 16, 16)] for k in range(_K1)]
            for j in range(16):
                e = gi * 16 + j
                v0, v1, v2, v3 = vv[0][j], vv[1][j], vv[2][j], vv[3][j]
                for g in range(_HID // 16):
                    o = g * 16
                    acc = rows_v[e, pl.ds(o, 16)] * v0
                    acc = acc + rows_v[e, pl.ds(_HID + o, 16)] * v1
                    acc = acc + rows_v[e, pl.ds(2 * _HID + o, 16)] * v2
                    acc = acc + rows_v[e, pl.ds(3 * _HID + o, 16)] * v3
                    msg_v[e, pl.ds(o, 16)] = acc

        pltpu.sync_copy(msg_v, msgs_hbm.at[pl.ds(base, _CHUNK)])
        return carry

    lax.fori_loop(0, _NCHUNKS, chunk_body, 0)


def _sc_msg_phase(y, col, vals):
    mesh = plsc.VectorSubcoreMesh(core_axis_name="c", subcore_axis_name="s")
    fn = functools.partial(
        pl.kernel,
        mesh=mesh,
        out_type=jax.ShapeDtypeStruct((_E_PAD, _HID), jnp.float32),
        scratch_types=[
            pltpu.VMEM((_CHUNK,), jnp.int32),
            pltpu.VMEM((_K1, _CHUNK), jnp.float32),
            pltpu.VMEM((_CHUNK, _YW), jnp.float32),
            pltpu.VMEM((_CHUNK, _HID), jnp.float32),
            pltpu.SemaphoreType.DMA,
        ],
    )(_sc_msg_kernel)
    return fn(y, col, vals)


def _sc_scatter_kernel(msgs_hbm, row_hbm, zeros_hbm, out_hbm,
                       rowv, rowv2, msgc_v, acc_sh, sem):
    c = lax.axis_index("c")
    s = lax.axis_index("s")
    wid = s * _NC + c
    base0 = wid * _EPW

    for p in range(_NP):
        pltpu.sync_copy(zeros_hbm, acc_sh.at[pl.ds(s * _ZR, _ZR)])
        plsc.subcore_barrier()

        def chunk_body(ci, carry):
            base = base0 + ci * _CHUNK
            pltpu.sync_copy(msgs_hbm.at[pl.ds(base, _CHUNK)], msgc_v)
            pltpu.sync_copy(row_hbm.at[pl.ds(base, _CHUNK)], rowv)
            for g in range(_CHUNK // 16):
                r = rowv[pl.ds(g * 16, 16)] - (p * _NSPLIT)
                in_range = (r >= 0) & (r < _NSPLIT)
                trash = _NSPLIT + (r & (_TRASH - 1))
                rowv2[pl.ds(g * 16, 16)] = jnp.where(in_range, r, trash)
            pltpu.sync_copy(msgc_v, acc_sh.at[rowv2], add=True)
            return carry

        lax.fori_loop(0, _NCHUNKS, chunk_body, 0)
        plsc.subcore_barrier()

        pltpu.sync_copy(acc_sh.at[pl.ds(s * (_NSPLIT // _NS), _NSPLIT // _NS)],
                        out_hbm.at[c, p, pl.ds(s * (_NSPLIT // _NS),
                                               _NSPLIT // _NS)])
        plsc.subcore_barrier()


def _sc_scatter_phase(msgs, row, zeros):
    mesh = plsc.VectorSubcoreMesh(core_axis_name="c", subcore_axis_name="s")
    fn = functools.partial(
        pl.kernel,
        mesh=mesh,
        out_type=jax.ShapeDtypeStruct((_NC, _NP, _NSPLIT, _HID), jnp.float32),
        scratch_types=[
            pltpu.VMEM((_CHUNK,), jnp.int32),
            pltpu.VMEM((_CHUNK,), jnp.int32),
            pltpu.VMEM((_CHUNK, _HID), jnp.float32),
            pltpu.VMEM_SHARED((_ACC_R, _HID), jnp.float32),
            pltpu.SemaphoreType.DMA,
        ],
    )(_sc_scatter_kernel)
    return fn(msgs, row, zeros)


def _combine_body(p0_ref, p1_ref, wb_ref, a_ref, out_ref):
    bias = jnp.sum(wb_ref[...], axis=0, keepdims=True)
    y = p0_ref[...] + p1_ref[...] + bias
    a = a_ref[0:1, :]
    out_ref[...] = jnp.where(y >= 0.0, y, a * y)


def _combine(p0, p1, wb8, a8):
    blk = 1000
    return pl.pallas_call(
        _combine_body,
        grid=(_N // blk,),
        in_specs=[
            pl.BlockSpec((blk, _HID), lambda i: (i, 0)),
            pl.BlockSpec((blk, _HID), lambda i: (i, 0)),
            pl.BlockSpec((8, _HID), lambda i: (0, 0)),
            pl.BlockSpec((8, _HID), lambda i: (0, 0)),
        ],
        out_specs=pl.BlockSpec((blk, _HID), lambda i: (i, 0)),
        out_shape=jax.ShapeDtypeStruct((_N, _HID), jnp.float32),
    )(p0, p1, wb8, a8)


def kernel(X, edge_index, edge_vals, W_w, W_b, prelu_a):
    wcat = jnp.transpose(W_w, (2, 0, 1)).reshape(_IN, _YW)
    pad = _E_PAD - _E
    row = jnp.pad(edge_index[0].astype(jnp.int32), (0, pad))
    col = jnp.pad(edge_index[1].astype(jnp.int32), (0, pad))
    vals = jnp.pad(edge_vals, ((0, 0), (0, pad))).reshape(-1)
    zeros = jnp.zeros((_ZR, _HID), jnp.float32)
    wb8 = jnp.pad(W_b, ((0, 8 - _K1), (0, 0)))
    a8 = jnp.full((8, _HID), prelu_a, jnp.float32)

    y = _compute_y(X, wcat)
    msgs = _sc_msg_phase(y, col, vals)
    partials = _sc_scatter_phase(msgs, row, zeros)
    partials = partials.reshape(_NC, _NP * _NSPLIT, _HID)
    return _combine(partials[0, :_N], partials[1, :_N], wb8, a8)
